# baseline (speedup 1.0000x reference)
import functools

import numpy as np

import jax
import jax.numpy as jnp
from jax import lax
from jax.experimental import pallas as pl
from jax.experimental.pallas import tpu as pltpu
from jax.experimental.pallas import tpu_sc as plsc

ALIGNED = 7
SR = 2
SCALE = 0.0625
B, C, H, W = 2, 256, 50, 50
N_ROIS = 1000
BINS = N_ROIS * ALIGNED * ALIGNED
LANES_PER_BIN = 16
ROI_BLOCK = 64
LANE_W = ALIGNED * ALIGNED * LANES_PER_BIN
PAIRS_PER_BIN = 8
LANE_I = ALIGNED * ALIGNED * PAIRS_PER_BIN

NW = 32
N_ROIS_PAD = 1024
TR_BLK = 50

_p = np.arange(C)
_COL_SRC = (_p // 32) * 32 + (_p % 2) * 16 + (_p % 32) // 2
CHUNK_BINS = 16
CHUNK_ROWS = CHUNK_BINS * PAIRS_PER_BIN
CHUNK_WLANES = CHUNK_BINS * LANES_PER_BIN
BINS_PAD = N_ROIS_PAD * ALIGNED * ALIGNED
CHUNKS_PER_W = BINS_PAD // (NW * CHUNK_BINS)
NCHUNKS = BINS_PAD // CHUNK_BINS


def _index_weight_body(rois_ref, idx_ref, w_ref):
    l = lax.broadcasted_iota(jnp.int32, (ROI_BLOCK, LANE_W), 1)
    py = l // (ALIGNED * LANES_PER_BIN)
    px = (l // LANES_PER_BIN) % ALIGNED
    q = l % LANES_PER_BIN
    a = q // 4
    b = q % 4
    iy = a // 2
    cy = a % 2
    ix = b // 2
    cx = b % 2

    def col(j):
        return rois_ref[:, j][:, None]

    bid = jnp.clip(col(0).astype(jnp.int32), 0, B - 1)
    x1 = col(1) * SCALE
    y1 = col(2) * SCALE
    x2 = col(3) * SCALE
    y2 = col(4) * SCALE
    roi_w = jnp.maximum(x2 - x1, 1.0)
    roi_h = jnp.maximum(y2 - y1, 1.0)
    bin_w = roi_w / ALIGNED
    bin_h = roi_h / ALIGNED

    y_s = y1 + py.astype(jnp.float32) * bin_h \
        + (iy.astype(jnp.float32) + 0.5) * bin_h / SR
    x_s = x1 + px.astype(jnp.float32) * bin_w \
        + (ix.astype(jnp.float32) + 0.5) * bin_w / SR

    my = (y_s > -1.0) & (y_s < float(H))
    mx = (x_s > -1.0) & (x_s < float(W))
    yc = jnp.clip(y_s, 0.0, float(H - 1))
    xc = jnp.clip(x_s, 0.0, float(W - 1))
    y_low = jnp.floor(yc)
    x_low = jnp.floor(xc)
    ly = yc - y_low
    lx = xc - x_low
    y_lo_i = y_low.astype(jnp.int32)
    x_lo_i = x_low.astype(jnp.int32)
    y_hi_i = jnp.minimum(y_lo_i + 1, H - 1)
    x_hi_i = jnp.minimum(x_lo_i + 1, W - 1)

    ysel = jnp.where(cy == 0, y_lo_i, y_hi_i)
    wy = jnp.where(cy == 0, 1.0 - ly, ly)
    wy = jnp.where(my, wy * 0.5, 0.0)
    wx0 = jnp.where(mx, (1.0 - lx) * 0.5, 0.0)
    wx1 = jnp.where(mx, lx * 0.5, 0.0)
    shift = x_lo_i == (W - 1)
    wa = jnp.where(shift, 0.0, wx0)
    wb = jnp.where(shift, wx0, wx1)
    wx = jnp.where(cx == 0, wa, wb)

    w_ref[...] = wy * wx
    li = lax.broadcasted_iota(jnp.int32, (ROI_BLOCK, LANE_I), 1)
    py_i = li // (ALIGNED * 8)
    px_i = (li // 8) % ALIGNED
    a_i = (li % 8) // 2
    ix_i = li % 2
    iy_i = a_i // 2
    cy_i = a_i % 2
    y_si = y1 + py_i.astype(jnp.float32) * bin_h \
        + (iy_i.astype(jnp.float32) + 0.5) * bin_h / SR
    x_si = x1 + px_i.astype(jnp.float32) * bin_w \
        + (ix_i.astype(jnp.float32) + 0.5) * bin_w / SR
    yc_i = jnp.clip(y_si, 0.0, float(H - 1))
    xc_i = jnp.clip(x_si, 0.0, float(W - 1))
    y_lo2 = jnp.floor(yc_i).astype(jnp.int32)
    x_lo2 = jnp.floor(xc_i).astype(jnp.int32)
    y_hi2 = jnp.minimum(y_lo2 + 1, H - 1)
    ysel_i = jnp.where(cy_i == 0, y_lo2, y_hi2)
    pstart = jnp.minimum(x_lo2, W - 2)
    idx_ref[...] = bid * (H * W) + ysel_i * W + pstart


def _compute_index_weights(rois):
    grid = N_ROIS_PAD // ROI_BLOCK
    return pl.pallas_call(
        _index_weight_body,
        grid=(grid,),
        in_specs=[pl.BlockSpec((ROI_BLOCK, 5), lambda i: (i, 0))],
        out_specs=[
            pl.BlockSpec((ROI_BLOCK, LANE_I), lambda i: (i, 0)),
            pl.BlockSpec((ROI_BLOCK, LANE_W), lambda i: (i, 0)),
        ],
        out_shape=[
            jax.ShapeDtypeStruct((N_ROIS_PAD, LANE_I), jnp.int32),
            jax.ShapeDtypeStruct((N_ROIS_PAD, LANE_W), jnp.float32),
        ],
    )(rois)


def _sc_gather_accumulate(table, idx, wts):
    mesh = plsc.VectorSubcoreMesh(core_axis_name="c", subcore_axis_name="s")

    @functools.partial(
        pl.kernel,
        mesh=mesh,
        out_type=jax.ShapeDtypeStruct((NCHUNKS, CHUNK_BINS, C), jnp.float32),
        scratch_types=[
            pltpu.VMEM((CHUNKS_PER_W, CHUNK_ROWS), jnp.int32),
            pltpu.VMEM((CHUNKS_PER_W, CHUNK_WLANES), jnp.float32),
            pltpu.VMEM((CHUNK_ROWS, C), jnp.int32),
            pltpu.VMEM((CHUNK_ROWS, C), jnp.int32),
            pltpu.VMEM((CHUNK_BINS, C), jnp.float32),
            pltpu.VMEM((CHUNK_BINS, C), jnp.float32),
            pltpu.SemaphoreType.DMA,
            pltpu.SemaphoreType.DMA,
            pltpu.SemaphoreType.DMA,
            pltpu.SemaphoreType.DMA,
        ],
    )
    def k(table_hbm, idx_hbm, w_hbm, out_hbm, idx_all, w_all,
          rows_a, rows_b, out_a, out_b, s_ga, s_gb, s_oa, s_ob):
        wid = lax.axis_index("s") * 2 + lax.axis_index("c")
        obase = wid * CHUNKS_PER_W
        pltpu.sync_copy(idx_hbm.at[wid], idx_all)
        pltpu.sync_copy(w_hbm.at[wid], w_all)

        def accumulate(g, rows_v, out_v):
            def tree(terms):
                while len(terms) > 1:
                    terms = [terms[j] + terms[j + 1]
                             for j in range(0, len(terms), 2)]
                return terms[0]

            def one_bin(bn):
                wvec = w_all[g, pl.ds(bn * LANES_PER_BIN, LANES_PER_BIN)]
                wsp = [wvec[i] for i in range(LANES_PER_BIN)]
                for grp in range(C // 32):
                    terms_e = []
                    terms_o = []
                    for p in range(PAIRS_PER_BIN):
                        for s in range(2):
                            v = rows_v[bn * PAIRS_PER_BIN + p,
                                       pl.ds(s * (C // 2) + grp * 16, 16)]
                            ve = lax.bitcast_convert_type(
                                lax.shift_left(v, 16), jnp.float32)
                            vo = lax.bitcast_convert_type(v, jnp.float32)
                            w_ps = wsp[p * 2 + s]
                            terms_e.append(w_ps * ve)
                            terms_o.append(w_ps * vo)
                    out_v[bn, pl.ds(grp * 32, 16)] = tree(terms_e)
                    out_v[bn, pl.ds(grp * 32 + 16, 16)] = tree(terms_o)

            @plsc.parallel_loop(0, CHUNK_BINS, 1, unroll=8)
            def bin_body(m):
                one_bin(m)

        def gather_start(g, rows_v, sem):
            pltpu.async_copy(table_hbm.at[idx_all.at[g]], rows_v, sem)

        def gather_wait(g, rows_v, sem):
            pltpu.make_async_copy(table_hbm.at[idx_all.at[g]], rows_v,
                                  sem).wait()

        def store_start(g, out_v, sem):
            pltpu.async_copy(out_v, out_hbm.at[obase + g], sem)

        def store_wait(out_v, sem):
            pltpu.make_async_copy(out_v, out_hbm.at[obase], sem).wait()

        gather_start(0, rows_a, s_ga)

        def body(kk, carry):
            g0 = 2 * kk
            g1 = g0 + 1
            gather_start(g1, rows_b, s_gb)
            gather_wait(g0, rows_a, s_ga)

            @pl.when(kk > 0)
            def _():
                store_wait(out_a, s_oa)

            accumulate(g0, rows_a, out_a)
            store_start(g0, out_a, s_oa)

            @pl.when(kk < CHUNKS_PER_W // 2 - 1)
            def _():
                gather_start(g0 + 2, rows_a, s_ga)

            gather_wait(g1, rows_b, s_gb)

            @pl.when(kk > 0)
            def _():
                store_wait(out_b, s_ob)

            accumulate(g1, rows_b, out_b)
            store_start(g1, out_b, s_ob)
            return carry

        lax.fori_loop(0, CHUNKS_PER_W // 2, body, 0)
        store_wait(out_a, s_oa)
        store_wait(out_b, s_ob)

    return k(table, idx, wts)


def _transpose_body(rows_ref, out_ref):
    out_ref[...] = jnp.transpose(rows_ref[...], (0, 2, 1))


def _transpose_rows(rows):
    grid = N_ROIS // TR_BLK
    return pl.pallas_call(
        _transpose_body,
        grid=(grid,),
        in_specs=[pl.BlockSpec((TR_BLK, ALIGNED * ALIGNED, C),
                               lambda i: (i, 0, 0))],
        out_specs=pl.BlockSpec((TR_BLK, C, ALIGNED * ALIGNED),
                               lambda i: (i, 0, 0)),
        out_shape=jax.ShapeDtypeStruct((N_ROIS, C, ALIGNED * ALIGNED),
                                       jnp.float32),
    )(rows)


def kernel(features, rois):
    idx, wts = _compute_index_weights(
        jnp.pad(rois, ((0, N_ROIS_PAD - N_ROIS), (0, 0))))

    table = jnp.transpose(features, (0, 2, 3, 1)).reshape(B * H * W, C)
    table = jnp.take(table, jnp.asarray(_COL_SRC), axis=1)
    packed = lax.bitcast_convert_type(
        table.astype(jnp.bfloat16).reshape(B * H * W, C // 2, 2), jnp.int32)
    nxt = jnp.concatenate(
        [packed[1:], jnp.zeros((1, C // 2), jnp.int32)], axis=0)
    table2 = jnp.concatenate([packed, nxt], axis=1)
    idx = idx.reshape(NW, CHUNKS_PER_W, CHUNK_ROWS)
    wts = wts.reshape(NW, CHUNKS_PER_W, CHUNK_WLANES)

    rows = _sc_gather_accumulate(table2, idx, wts)
    rows = rows.reshape(BINS_PAD, C)[:BINS]
    out = _transpose_rows(rows.reshape(N_ROIS, ALIGNED * ALIGNED, C))
    return out.reshape(N_ROIS, C, ALIGNED, ALIGNED)


# speedup vs baseline: 1.2483x; 1.2483x over previous
import functools

import numpy as np

import jax
import jax.numpy as jnp
from jax import lax
from jax.experimental import pallas as pl
from jax.experimental.pallas import tpu as pltpu
from jax.experimental.pallas import tpu_sc as plsc

ALIGNED = 7
SR = 2
SCALE = 0.0625
B, C, H, W = 2, 256, 50, 50
N_ROIS = 1000
BINS = N_ROIS * ALIGNED * ALIGNED
LANES_PER_BIN = 16
ROI_BLOCK = 8
LANE_W = ALIGNED * ALIGNED * LANES_PER_BIN

NW = 32
N_ROIS_PAD = 1024
TR_BLK = 8

_p = np.arange(C)
_COL_SRC = (_p // 32) * 32 + (_p % 2) * 16 + (_p % 32) // 2
CHUNK_BINS = 8
CHUNK_ROWS = CHUNK_BINS * LANES_PER_BIN
BINS_PAD = N_ROIS_PAD * ALIGNED * ALIGNED
CHUNKS_PER_W = BINS_PAD // (NW * CHUNK_BINS)
NCHUNKS = BINS_PAD // CHUNK_BINS


def _index_weight_body(rois_ref, idx_ref, w_ref):
    l = lax.broadcasted_iota(jnp.int32, (ROI_BLOCK, LANE_W), 1)
    py = l // (ALIGNED * LANES_PER_BIN)
    px = (l // LANES_PER_BIN) % ALIGNED
    q = l % LANES_PER_BIN
    a = q // 4
    b = q % 4
    iy = a // 2
    cy = a % 2
    ix = b // 2
    cx = b % 2

    def col(j):
        return rois_ref[:, j][:, None]

    bid = jnp.clip(col(0).astype(jnp.int32), 0, B - 1)
    x1 = col(1) * SCALE
    y1 = col(2) * SCALE
    x2 = col(3) * SCALE
    y2 = col(4) * SCALE
    roi_w = jnp.maximum(x2 - x1, 1.0)
    roi_h = jnp.maximum(y2 - y1, 1.0)
    bin_w = roi_w / ALIGNED
    bin_h = roi_h / ALIGNED

    y_s = y1 + py.astype(jnp.float32) * bin_h \
        + (iy.astype(jnp.float32) + 0.5) * bin_h / SR
    x_s = x1 + px.astype(jnp.float32) * bin_w \
        + (ix.astype(jnp.float32) + 0.5) * bin_w / SR

    my = (y_s > -1.0) & (y_s < float(H))
    mx = (x_s > -1.0) & (x_s < float(W))
    yc = jnp.clip(y_s, 0.0, float(H - 1))
    xc = jnp.clip(x_s, 0.0, float(W - 1))
    y_low = jnp.floor(yc)
    x_low = jnp.floor(xc)
    ly = yc - y_low
    lx = xc - x_low
    y_lo_i = y_low.astype(jnp.int32)
    x_lo_i = x_low.astype(jnp.int32)
    y_hi_i = jnp.minimum(y_lo_i + 1, H - 1)
    x_hi_i = jnp.minimum(x_lo_i + 1, W - 1)

    ysel = jnp.where(cy == 0, y_lo_i, y_hi_i)
    xsel = jnp.where(cx == 0, x_lo_i, x_hi_i)
    wy = jnp.where(cy == 0, 1.0 - ly, ly)
    wx = jnp.where(cx == 0, 1.0 - lx, lx)
    wy = jnp.where(my, wy * 0.5, 0.0)
    wx = jnp.where(mx, wx * 0.5, 0.0)

    idx_ref[...] = bid * (H * W) + ysel * W + xsel
    w_ref[...] = wy * wx


def _compute_index_weights(rois):
    grid = N_ROIS_PAD // ROI_BLOCK
    return pl.pallas_call(
        _index_weight_body,
        grid=(grid,),
        in_specs=[pl.BlockSpec((ROI_BLOCK, 5), lambda i: (i, 0))],
        out_specs=[
            pl.BlockSpec((ROI_BLOCK, LANE_W), lambda i: (i, 0)),
            pl.BlockSpec((ROI_BLOCK, LANE_W), lambda i: (i, 0)),
        ],
        out_shape=[
            jax.ShapeDtypeStruct((N_ROIS_PAD, LANE_W), jnp.int32),
            jax.ShapeDtypeStruct((N_ROIS_PAD, LANE_W), jnp.float32),
        ],
    )(rois)


def _sc_gather_accumulate(table, idx, wts):
    mesh = plsc.VectorSubcoreMesh(core_axis_name="c", subcore_axis_name="s")

    @functools.partial(
        pl.kernel,
        mesh=mesh,
        out_type=jax.ShapeDtypeStruct((NCHUNKS, CHUNK_BINS, C), jnp.float32),
        scratch_types=[
            pltpu.VMEM((CHUNKS_PER_W, CHUNK_ROWS), jnp.int32),
            pltpu.VMEM((CHUNKS_PER_W, CHUNK_ROWS), jnp.float32),
            pltpu.VMEM((CHUNK_ROWS, C // 2), jnp.int32),
            pltpu.VMEM((CHUNK_ROWS, C // 2), jnp.int32),
            pltpu.VMEM((CHUNK_BINS, C), jnp.float32),
            pltpu.VMEM((CHUNK_BINS, C), jnp.float32),
            pltpu.SemaphoreType.DMA,
            pltpu.SemaphoreType.DMA,
            pltpu.SemaphoreType.DMA,
            pltpu.SemaphoreType.DMA,
        ],
    )
    def k(table_hbm, idx_hbm, w_hbm, out_hbm, idx_all, w_all,
          rows_a, rows_b, out_a, out_b, s_ga, s_gb, s_oa, s_ob):
        wid = lax.axis_index("s") * 2 + lax.axis_index("c")
        obase = wid * CHUNKS_PER_W
        pltpu.sync_copy(idx_hbm.at[wid], idx_all)
        pltpu.sync_copy(w_hbm.at[wid], w_all)

        def accumulate(g, rows_v, out_v):
            def tree(terms):
                while len(terms) > 1:
                    terms = [terms[j] + terms[j + 1]
                             for j in range(0, len(terms), 2)]
                return terms[0]

            def one_bin(bn):
                wvec = w_all[g, pl.ds(bn * LANES_PER_BIN, LANES_PER_BIN)]
                wsp = [wvec[i] for i in range(LANES_PER_BIN)]
                for grp in range(C // 32):
                    terms_e = []
                    terms_o = []
                    for i in range(LANES_PER_BIN):
                        v = rows_v[bn * LANES_PER_BIN + i,
                                   pl.ds(grp * 16, 16)]
                        ve = lax.bitcast_convert_type(
                            lax.shift_left(v, 16), jnp.float32)
                        vo = lax.bitcast_convert_type(v, jnp.float32)
                        terms_e.append(wsp[i] * ve)
                        terms_o.append(wsp[i] * vo)
                    out_v[bn, pl.ds(grp * 32, 16)] = tree(terms_e)
                    out_v[bn, pl.ds(grp * 32 + 16, 16)] = tree(terms_o)

            def bin_body(m, carry):
                one_bin(2 * m)
                one_bin(2 * m + 1)
                return carry

            lax.fori_loop(0, CHUNK_BINS // 2, bin_body, 0)

        def gather_start(g, rows_v, sem):
            pltpu.async_copy(table_hbm.at[idx_all.at[g]], rows_v, sem)

        def gather_wait(g, rows_v, sem):
            pltpu.make_async_copy(table_hbm.at[idx_all.at[g]], rows_v,
                                  sem).wait()

        def store_start(g, out_v, sem):
            pltpu.async_copy(out_v, out_hbm.at[obase + g], sem)

        def store_wait(out_v, sem):
            pltpu.make_async_copy(out_v, out_hbm.at[obase], sem).wait()

        gather_start(0, rows_a, s_ga)

        def body(kk, carry):
            g0 = 2 * kk
            g1 = g0 + 1
            gather_start(g1, rows_b, s_gb)
            gather_wait(g0, rows_a, s_ga)

            @pl.when(kk > 0)
            def _():
                store_wait(out_a, s_oa)

            accumulate(g0, rows_a, out_a)
            store_start(g0, out_a, s_oa)

            @pl.when(kk < CHUNKS_PER_W // 2 - 1)
            def _():
                gather_start(g0 + 2, rows_a, s_ga)

            gather_wait(g1, rows_b, s_gb)

            @pl.when(kk > 0)
            def _():
                store_wait(out_b, s_ob)

            accumulate(g1, rows_b, out_b)
            store_start(g1, out_b, s_ob)
            return carry

        lax.fori_loop(0, CHUNKS_PER_W // 2, body, 0)
        store_wait(out_a, s_oa)
        store_wait(out_b, s_ob)

    return k(table, idx, wts)


def _transpose_body(rows_ref, out_ref):
    out_ref[...] = jnp.transpose(rows_ref[...], (0, 2, 1))


def _transpose_rows(rows):
    grid = N_ROIS // TR_BLK
    return pl.pallas_call(
        _transpose_body,
        grid=(grid,),
        in_specs=[pl.BlockSpec((TR_BLK, ALIGNED * ALIGNED, C),
                               lambda i: (i, 0, 0))],
        out_specs=pl.BlockSpec((TR_BLK, C, ALIGNED * ALIGNED),
                               lambda i: (i, 0, 0)),
        out_shape=jax.ShapeDtypeStruct((N_ROIS, C, ALIGNED * ALIGNED),
                                       jnp.float32),
    )(rows)


def kernel(features, rois):
    idx, wts = _compute_index_weights(
        jnp.pad(rois, ((0, N_ROIS_PAD - N_ROIS), (0, 0))))

    table = jnp.transpose(features, (0, 2, 3, 1)).reshape(B * H * W, C)
    table = jnp.take(table, jnp.asarray(_COL_SRC), axis=1)
    table = lax.bitcast_convert_type(
        table.astype(jnp.bfloat16).reshape(B * H * W, C // 2, 2), jnp.int32)
    idx = idx.reshape(NW, CHUNKS_PER_W, CHUNK_ROWS)
    wts = wts.reshape(NW, CHUNKS_PER_W, CHUNK_ROWS)

    rows = _sc_gather_accumulate(table, idx, wts)
    rows = rows.reshape(BINS_PAD, C)[:BINS]
    out = _transpose_rows(rows.reshape(N_ROIS, ALIGNED * ALIGNED, C))
    return out.reshape(N_ROIS, C, ALIGNED, ALIGNED)
